# Initial kernel scaffold; baseline (speedup 1.0000x reference)
import jax
import jax.numpy as jnp
from jax.experimental import pallas as pl


def kernel(target, x):
    raise NotImplementedError("write your pallas kernel here")


# speedup vs baseline: 12.1569x; 12.1569x over previous
import jax
import jax.numpy as jnp
from jax import lax
from jax.experimental import pallas as pl
from jax.experimental.pallas import tpu as pltpu
from jax.experimental.pallas import tpu_sc as plsc

N = 65536
D = 32
K = 64
KM_ITERS = 5
BLK = 8192
NBLK = N // BLK


def _dist2_labels(xb, c):
    x2 = jnp.sum(xb * xb, axis=1, keepdims=True)
    c2 = lax.dot_general(jnp.ones((1, D), jnp.float32), c * c,
                         (((1,), (1,)), ((), ())),
                         preferred_element_type=jnp.float32)
    xc = lax.dot_general(xb, c, (((1,), (1,)), ((), ())),
                         preferred_element_type=jnp.float32)
    d2 = jnp.maximum(x2 + c2 - 2.0 * xc, 1e-12)
    dmin = jnp.min(d2, axis=1, keepdims=True)
    cols = lax.broadcasted_iota(jnp.int32, d2.shape, 1)
    lbl = jnp.min(jnp.where(d2 == dmin, cols, K), axis=1, keepdims=True)
    return d2, lbl


def _kmeans_step_kernel(x_ref, c_ref, newc_ref, acc_sum, acc_cnt):
    i = pl.program_id(0)

    @pl.when(i == 0)
    def _():
        acc_sum[...] = jnp.zeros_like(acc_sum)
        acc_cnt[...] = jnp.zeros_like(acc_cnt)

    xb = x_ref[...]
    c = c_ref[...]
    _, lbl = _dist2_labels(xb, c)
    cols = lax.broadcasted_iota(jnp.int32, (BLK, K), 1)
    oh = (cols == lbl).astype(jnp.float32)
    acc_sum[...] += lax.dot_general(oh, xb, (((0,), (0,)), ((), ())),
                                    preferred_element_type=jnp.float32)
    acc_cnt[...] += lax.dot_general(oh, jnp.ones((BLK, 1), jnp.float32),
                                    (((0,), (0,)), ((), ())),
                                    preferred_element_type=jnp.float32)

    @pl.when(i == pl.num_programs(0) - 1)
    def _():
        newc_ref[...] = acc_sum[...] / jnp.maximum(acc_cnt[...], 1.0)


def _kmeans_step(x, c):
    return pl.pallas_call(
        _kmeans_step_kernel,
        grid=(NBLK,),
        in_specs=[
            pl.BlockSpec((BLK, D), lambda i: (i, 0)),
            pl.BlockSpec((K, D), lambda i: (0, 0)),
        ],
        out_specs=pl.BlockSpec((K, D), lambda i: (0, 0)),
        out_shape=jax.ShapeDtypeStruct((K, D), jnp.float32),
        scratch_shapes=[
            pltpu.VMEM((K, D), jnp.float32),
            pltpu.VMEM((K, 1), jnp.float32),
        ],
    )(x, c)


def _softmax_colsum(d2):
    dist = jnp.sqrt(d2)
    z = -dist
    z = z - jnp.max(z, axis=1, keepdims=True)
    e = jnp.exp(z)
    p = e / jnp.sum(e, axis=1, keepdims=True)
    return lax.dot_general(jnp.ones((1, BLK), jnp.float32), p,
                           (((1,), (0,)), ((), ())),
                           preferred_element_type=jnp.float32)


def _assign_kernel(t_ref, x_ref, c_ref, lt_ref, lx_ref, lf_ref, cnt_ref,
                   aft, afx, act, acx):
    i = pl.program_id(0)

    @pl.when(i == 0)
    def _():
        aft[...] = jnp.zeros_like(aft)
        afx[...] = jnp.zeros_like(afx)
        act[...] = jnp.zeros_like(act)
        acx[...] = jnp.zeros_like(acx)

    c = c_ref[...]
    d2t, lt = _dist2_labels(t_ref[...], c)
    d2x, lx = _dist2_labels(x_ref[...], c)
    lt_ref[...] = lt.reshape(1, 1, BLK)
    lx_ref[...] = lx.reshape(1, 1, BLK)
    aft[...] += _softmax_colsum(d2t)
    afx[...] += _softmax_colsum(d2x)
    cols = lax.broadcasted_iota(jnp.int32, (BLK, K), 1)
    ones_row = jnp.ones((1, BLK), jnp.float32)
    oh_t = (cols == lt).astype(jnp.float32)
    oh_x = (cols == lx).astype(jnp.float32)
    act[...] += lax.dot_general(ones_row, oh_t, (((1,), (0,)), ((), ())),
                                preferred_element_type=jnp.float32)
    acx[...] += lax.dot_general(ones_row, oh_x, (((1,), (0,)), ((), ())),
                                preferred_element_type=jnp.float32)

    @pl.when(i == pl.num_programs(0) - 1)
    def _():
        ft = aft[...] / float(N)
        fx = afx[...] / float(N)
        lf_ref[...] = jnp.sum((fx - ft) ** 2, keepdims=True) / float(K)
        cnt_ref[...] = jnp.concatenate(
            [act[...], acx[...]], axis=1).astype(jnp.int32)


def _assign(t, x, c):
    return pl.pallas_call(
        _assign_kernel,
        grid=(NBLK,),
        in_specs=[
            pl.BlockSpec((BLK, D), lambda i: (i, 0)),
            pl.BlockSpec((BLK, D), lambda i: (i, 0)),
            pl.BlockSpec((K, D), lambda i: (0, 0)),
        ],
        out_specs=[
            pl.BlockSpec((1, 1, BLK), lambda i: (i, 0, 0)),
            pl.BlockSpec((1, 1, BLK), lambda i: (i, 0, 0)),
            pl.BlockSpec((1, 1), lambda i: (0, 0)),
            pl.BlockSpec((1, 2 * K), lambda i: (0, 0)),
        ],
        out_shape=[
            jax.ShapeDtypeStruct((NBLK, 1, BLK), jnp.int32),
            jax.ShapeDtypeStruct((NBLK, 1, BLK), jnp.int32),
            jax.ShapeDtypeStruct((1, 1), jnp.float32),
            jax.ShapeDtypeStruct((1, 2 * K), jnp.int32),
        ],
        scratch_shapes=[
            pltpu.VMEM((1, K), jnp.float32),
            pltpu.VMEM((1, K), jnp.float32),
            pltpu.VMEM((1, K), jnp.float32),
            pltpu.VMEM((1, K), jnp.float32),
        ],
    )(t, x, c)


def _bitonic1(Kv, R):
    def outer(kk, Kc):
        bs = lax.shift_left(jnp.int32(1), kk)
        asc = (R & bs) == 0

        def inner(t, Kc):
            j = lax.shift_right_logical(bs, t + 1)
            low = (R & j) == 0
            PK = jnp.where(low, pltpu.roll(Kc, N - j, 1), pltpu.roll(Kc, j, 1))
            cond = asc == low
            repl = (cond & (Kc > PK)) | (~cond & (Kc < PK))
            return jnp.where(repl, PK, Kc)

        return lax.fori_loop(0, kk, inner, Kc)

    return lax.fori_loop(1, 17, outer, Kv)


def _selkeys_kernel(lt_ref, lx_ref, cnt_ref, kt_ref, kx_ref):
    cnt = cnt_ref[...]
    ct = cnt[:, :K]
    cx = cnt[:, K:]
    m = jnp.minimum(ct, cx)
    R = lax.broadcasted_iota(jnp.int32, (1, N), 1)
    l64 = lax.broadcasted_iota(jnp.int32, (1, K), 1)

    def side(lbl, cnts, out_ref):
        c = cnts
        for s in (1, 2, 4, 8, 16, 32):
            c = c + jnp.where(l64 >= s, pltpu.roll(c, s, 1), 0)
        ss = c - cnts
        key1 = _bitonic1(lbl * (2 * N) + R, R)
        lbls = lax.shift_right_logical(key1, 17)
        idx = key1 & (2 * N - 1)
        sel = jnp.zeros((1, N), jnp.int32)
        for k in range(K):
            ok = (lbls == k) & (R - ss[:, k:k + 1] < m[:, k:k + 1])
            sel = jnp.where(ok, 1, sel)
        key2 = _bitonic1(idx * 256 + sel * 128 + lbls, R)
        lbl2 = key2 & 127
        sel2 = lax.shift_right_logical(key2, 7) & 1
        out_ref[...] = jnp.where(sel2 == 1, lbl2.astype(jnp.float32),
                                 float(K))

    side(lt_ref[...], ct, kt_ref)
    side(lx_ref[...], cx, kx_ref)


def _selkeys(lt, lx, cnt):
    return pl.pallas_call(
        _selkeys_kernel,
        out_shape=[
            jax.ShapeDtypeStruct((1, N), jnp.float32),
            jax.ShapeDtypeStruct((1, N), jnp.float32),
        ],
    )(lt, lx, cnt)


def _bitonic_lex(Kv, Vv, R):
    def outer(kk, carry):
        bs = lax.shift_left(jnp.int32(1), kk)
        asc = (R & bs) == 0

        def inner(t, carry):
            Kc, Vc = carry
            j = lax.shift_right_logical(bs, t + 1)
            low = (R & j) == 0
            PK = jnp.where(low, pltpu.roll(Kc, N - j, 1), pltpu.roll(Kc, j, 1))
            PV = jnp.where(low, pltpu.roll(Vc, N - j, 1), pltpu.roll(Vc, j, 1))
            gt = (Kc > PK) | ((Kc == PK) & (Vc > PV))
            lt = (Kc < PK) | ((Kc == PK) & (Vc < PV))
            cond = jnp.broadcast_to(asc == low, gt.shape)
            repl = (cond & gt) | (~cond & lt)
            return jnp.where(repl, PK, Kc), jnp.where(repl, PV, Vc)

        return lax.fori_loop(0, kk, inner, carry)

    return lax.fori_loop(1, 17, outer, (Kv, Vv))


CHUNK = 8
NCH = D // CHUNK


def _sort_reduce_kernel(kt_ref, kx_ref, cnt_ref, lf_ref,
                        tT_ref, xT_ref, out_ref, s_acc):
    i = pl.program_id(0)

    def minfo():
        cnt = cnt_ref[...]
        m = jnp.minimum(cnt[:, :K], cnt[:, K:])
        l64 = lax.broadcasted_iota(jnp.int32, (1, K), 1)
        c = m
        for s in (1, 2, 4, 8, 16, 32):
            c = c + jnp.where(l64 >= s, pltpu.roll(c, s, 1), 0)
        return m, c - m

    @pl.when(i == 0)
    def _():
        s_acc[...] = jnp.zeros_like(s_acc)

    R = lax.broadcasted_iota(jnp.int32, (1, N), 1)
    _, Vt = _bitonic_lex(jnp.broadcast_to(kt_ref[...], (CHUNK, N)),
                         tT_ref[...], R)
    _, Vx = _bitonic_lex(jnp.broadcast_to(kx_ref[...], (CHUNK, N)),
                         xT_ref[...], R)
    s_acc[...] += jnp.sum(jnp.abs(Vt - Vx), axis=0, keepdims=True)

    @pl.when(i == pl.num_programs(0) - 1)
    def _():
        m, moff = minfo()
        w = jnp.zeros((1, N), jnp.float32)
        for k in range(K):
            mk = m[:, k:k + 1]
            mok = moff[:, k:k + 1]
            mask = (R >= mok) & (R < mok + mk)
            wk = jnp.where(mk > 0,
                           1.0 / (float(D) *
                                  jnp.maximum(mk, 1).astype(jnp.float32)),
                           0.0)
            w = w + jnp.where(mask, wk, 0.0)
        out_ref[...] = jnp.sum(s_acc[...] * w, keepdims=True) + lf_ref[...]


def _sort_reduce(kt, kx, cnt, lf, tT, xT):
    return pl.pallas_call(
        _sort_reduce_kernel,
        grid=(NCH,),
        in_specs=[
            pl.BlockSpec((1, N), lambda i: (0, 0)),
            pl.BlockSpec((1, N), lambda i: (0, 0)),
            pl.BlockSpec((1, 2 * K), lambda i: (0, 0)),
            pl.BlockSpec((1, 1), lambda i: (0, 0)),
            pl.BlockSpec((CHUNK, N), lambda i: (i, 0)),
            pl.BlockSpec((CHUNK, N), lambda i: (i, 0)),
        ],
        out_specs=pl.BlockSpec((1, 1), lambda i: (0, 0)),
        out_shape=jax.ShapeDtypeStruct((1, 1), jnp.float32),
        scratch_shapes=[
            pltpu.VMEM((1, N), jnp.float32),
        ],
    )(kt, kx, cnt, lf, tT, xT)


def kernel(target, x):
    c = target[:K]
    for _ in range(KM_ITERS):
        c = _kmeans_step(target, c)
    lt3, lx3, loss_fil, cnt = _assign(target, x, c)
    kt, kx = _selkeys(lt3.reshape(1, N), lx3.reshape(1, N), cnt)
    out = _sort_reduce(kt, kx, cnt, loss_fil, target.T, x.T)
    return out.reshape(())
